# baseline (speedup 1.0000x reference)
import functools

import jax
import jax.numpy as jnp
from jax import lax
from jax.experimental import pallas as pl
from jax.experimental.pallas import tpu as pltpu
from jax.experimental.pallas import tpu_sc as plsc

NC = 2
NS = 16
NW = NC * NS
L = 16

CHUNK = 8192
UNROLL = 3
N_BINS = 10
HIST = N_BINS * L
HIST_PAD = 256


def _make_sc_call(n):
    fr = n // (NW * CHUNK)
    base_rem = fr * NW * CHUNK
    rem = n - base_rem
    mch = (rem // NW) // L * L
    tail2 = rem - mch * NW
    assert fr >= 2 and fr % 2 == 1 and mch > 0
    assert mch % 8 == 0 and tail2 % L == 0 and base_rem % 8 == 0

    mesh = plsc.VectorSubcoreMesh(core_axis_name="c", subcore_axis_name="s")

    @functools.partial(
        pl.kernel,
        mesh=mesh,
        compiler_params=pltpu.CompilerParams(needs_layout_passes=False),
        out_type=jax.ShapeDtypeStruct((NW * 3 * HIST_PAD,), jnp.float32),
        scratch_types=[
            pltpu.VMEM((CHUNK,), jnp.float32),
            pltpu.VMEM((CHUNK,), jnp.float32),
            pltpu.VMEM((CHUNK,), jnp.float32),
            pltpu.VMEM((CHUNK,), jnp.float32),
            pltpu.VMEM((HIST_PAD,), jnp.float32),
            pltpu.VMEM((HIST_PAD,), jnp.float32),
            pltpu.VMEM((HIST_PAD,), jnp.float32),
            pltpu.SemaphoreType.DMA,
            pltpu.SemaphoreType.DMA,
            pltpu.SemaphoreType.DMA,
            pltpu.SemaphoreType.DMA,
        ],
    )
    def sc_call(conf_hbm, risk_hbm, out_hbm, cbuf0, cbuf1, rbuf0, rbuf1,
                hcnt, hconf, hrisk, csem0, csem1, rsem0, rsem1):
        wid = lax.axis_index("s") * NC + lax.axis_index("c")
        cbufs = (cbuf0, cbuf1)
        rbufs = (rbuf0, rbuf1)
        csems = (csem0, csem1)
        rsems = (rsem0, rsem1)

        zero = jnp.zeros((L,), jnp.float32)
        for i in range(HIST_PAD // L):
            hcnt[pl.ds(i * L, L)] = zero
            hconf[pl.ds(i * L, L)] = zero
            hrisk[pl.ds(i * L, L)] = zero

        lane = lax.iota(jnp.int32, L)
        ones = jnp.ones((L,), jnp.float32)
        kvec = jnp.full((L,), 10.0 - 2.0 ** -20, jnp.float32)
        four = jnp.full((L,), 4, jnp.int32)
        oneu = jnp.ones((L,), jnp.uint32)
        ubound = jnp.full((L,), 0x411FFFFF, jnp.uint32)

        def scatter_one(cc, rr):
            t = cc * kvec
            bi = t.astype(jnp.int32)
            sh = lax.shift_left(bi, four)
            tb = lax.bitcast_convert_type(t, jnp.uint32)
            valid = (tb - oneu) < ubound
            idx = sh | lane
            plsc.addupdate_scatter(hcnt, [idx], ones, mask=valid)
            plsc.addupdate_scatter(hconf, [idx], cc, mask=valid)
            plsc.addupdate_scatter(hrisk, [idx], rr, mask=valid)

        def run_accum(cb, rb, nvregs):
            @plsc.parallel_loop(0, nvregs, unroll=UNROLL)
            def _(i):
                scatter_one(cb[pl.ds(i * L, L)], rb[pl.ds(i * L, L)])

        def start_full(k, b):
            off = (k * NW + wid) * CHUNK
            pltpu.async_copy(conf_hbm.at[pl.ds(off, CHUNK)], cbufs[b], csems[b])
            pltpu.async_copy(risk_hbm.at[pl.ds(off, CHUNK)], rbufs[b], rsems[b])

        def wait_full(k, b):
            off = (k * NW + wid) * CHUNK
            pltpu.make_async_copy(
                conf_hbm.at[pl.ds(off, CHUNK)], cbufs[b], csems[b]).wait()
            pltpu.make_async_copy(
                risk_hbm.at[pl.ds(off, CHUNK)], rbufs[b], rsems[b]).wait()

        start_full(0, 0)
        start_full(1, 1)

        def outer(g, _):
            k = 2 * g
            wait_full(k, 0)
            run_accum(cbuf0, rbuf0, CHUNK // L)

            @pl.when(k + 2 < fr)
            def _():
                start_full(k + 2, 0)

            wait_full(k + 1, 1)
            run_accum(cbuf1, rbuf1, CHUNK // L)

            @pl.when(k + 3 < fr)
            def _():
                start_full(k + 3, 1)

            return 0

        lax.fori_loop(0, fr // 2, outer, 0)

        moff = base_rem + wid * mch
        pltpu.async_copy(conf_hbm.at[pl.ds(moff, mch)],
                         cbuf1.at[pl.ds(0, mch)], csem1)
        pltpu.async_copy(risk_hbm.at[pl.ds(moff, mch)],
                         rbuf1.at[pl.ds(0, mch)], rsem1)
        wait_full(fr - 1, 0)
        run_accum(cbuf0, rbuf0, CHUNK // L)
        pltpu.make_async_copy(conf_hbm.at[pl.ds(moff, mch)],
                              cbuf1.at[pl.ds(0, mch)], csem1).wait()
        pltpu.make_async_copy(risk_hbm.at[pl.ds(moff, mch)],
                              rbuf1.at[pl.ds(0, mch)], rsem1).wait()
        run_accum(cbuf1, rbuf1, mch // L)

        if tail2:
            toff = base_rem + NW * mch

            @pl.when(wid == NW - 1)
            def _():
                pltpu.sync_copy(conf_hbm.at[pl.ds(toff, tail2)],
                                cbuf0.at[pl.ds(0, tail2)])
                pltpu.sync_copy(risk_hbm.at[pl.ds(toff, tail2)],
                                rbuf0.at[pl.ds(0, tail2)])
                run_accum(cbuf0, rbuf0, tail2 // L)

        base = wid * (3 * HIST_PAD)
        pltpu.sync_copy(hcnt, out_hbm.at[pl.ds(base, HIST_PAD)])
        pltpu.sync_copy(hconf, out_hbm.at[pl.ds(base + HIST_PAD, HIST_PAD)])
        pltpu.sync_copy(hrisk, out_hbm.at[pl.ds(base + 2 * HIST_PAD, HIST_PAD)])

    return sc_call


def kernel(confidences, risk):
    n = confidences.shape[0]
    p = _make_sc_call(n)(confidences, risk)
    s = p.reshape(NW, 3, 2, 128).sum(axis=0)
    s = s.reshape(3, HIST_PAD)[:, :HIST].reshape(3, N_BINS, L).sum(axis=2)
    cnt = s[0]
    sum_conf = s[1]
    sum_risk = cnt - s[2]
    safe = jnp.maximum(cnt, 1.0)
    contrib = jnp.abs(sum_conf / safe - sum_risk / safe) * (cnt / n)
    ece = jnp.sum(jnp.where(cnt > 0.0, contrib, 0.0))
    return ece.reshape((1,))


# speedup vs baseline: 1.0430x; 1.0430x over previous
import functools

import jax
import jax.numpy as jnp
from jax import lax
from jax.experimental import pallas as pl
from jax.experimental.pallas import tpu as pltpu
from jax.experimental.pallas import tpu_sc as plsc

NC = 2
NS = 16
NW = NC * NS
L = 16

CHUNK = 8192
UNROLL = 4
N_BINS = 10
HIST = N_BINS * L
HIST_PAD = 256


def _make_sc_call(n):
    fr = n // (NW * CHUNK)
    base_rem = fr * NW * CHUNK
    rem = n - base_rem
    mch = (rem // NW) // L * L
    tail2 = rem - mch * NW
    assert fr >= 2 and fr % 2 == 1 and mch > 0
    assert mch % 8 == 0 and tail2 % L == 0 and base_rem % 8 == 0

    mesh = plsc.VectorSubcoreMesh(core_axis_name="c", subcore_axis_name="s")

    @functools.partial(
        pl.kernel,
        mesh=mesh,
        compiler_params=pltpu.CompilerParams(needs_layout_passes=False),
        out_type=jax.ShapeDtypeStruct((NW * 3 * HIST_PAD,), jnp.float32),
        scratch_types=[
            pltpu.VMEM((CHUNK,), jnp.float32),
            pltpu.VMEM((CHUNK,), jnp.float32),
            pltpu.VMEM((CHUNK,), jnp.float32),
            pltpu.VMEM((CHUNK,), jnp.float32),
            pltpu.VMEM((HIST_PAD,), jnp.float32),
            pltpu.VMEM((HIST_PAD,), jnp.float32),
            pltpu.VMEM((HIST_PAD,), jnp.float32),
            pltpu.SemaphoreType.DMA,
            pltpu.SemaphoreType.DMA,
            pltpu.SemaphoreType.DMA,
            pltpu.SemaphoreType.DMA,
        ],
    )
    def sc_call(conf_hbm, risk_hbm, out_hbm, cbuf0, cbuf1, rbuf0, rbuf1,
                hcnt, hconf, hrisk, csem0, csem1, rsem0, rsem1):
        wid = lax.axis_index("s") * NC + lax.axis_index("c")
        cbufs = (cbuf0, cbuf1)
        rbufs = (rbuf0, rbuf1)
        csems = (csem0, csem1)
        rsems = (rsem0, rsem1)

        zero = jnp.zeros((L,), jnp.float32)
        for i in range(HIST_PAD // L):
            hcnt[pl.ds(i * L, L)] = zero
            hconf[pl.ds(i * L, L)] = zero
            hrisk[pl.ds(i * L, L)] = zero

        lane = lax.iota(jnp.int32, L)
        ones = jnp.ones((L,), jnp.float32)
        kvec = jnp.full((L,), 10.0 - 2.0 ** -20, jnp.float32)
        four = jnp.full((L,), 4, jnp.int32)
        oneu = jnp.ones((L,), jnp.uint32)
        ubound = jnp.full((L,), 0x411FFFFF, jnp.uint32)

        def scatter_one(cc, rr):
            t = cc * kvec
            bi = t.astype(jnp.int32)
            sh = lax.shift_left(bi, four)
            tb = lax.bitcast_convert_type(t, jnp.uint32)
            valid = (tb - oneu) < ubound
            idx = sh | lane
            plsc.addupdate_scatter(hcnt, [idx], ones, mask=valid)
            plsc.addupdate_scatter(hconf, [idx], cc, mask=valid)
            plsc.addupdate_scatter(hrisk, [idx], rr, mask=valid)

        def run_accum(cb, rb, nvregs):
            @plsc.parallel_loop(0, nvregs, unroll=UNROLL)
            def _(i):
                scatter_one(cb[pl.ds(i * L, L)], rb[pl.ds(i * L, L)])

        def start_full(k, b):
            off = (k * NW + wid) * CHUNK
            pltpu.async_copy(conf_hbm.at[pl.ds(off, CHUNK)], cbufs[b], csems[b])
            pltpu.async_copy(risk_hbm.at[pl.ds(off, CHUNK)], rbufs[b], rsems[b])

        def wait_full(k, b):
            off = (k * NW + wid) * CHUNK
            pltpu.make_async_copy(
                conf_hbm.at[pl.ds(off, CHUNK)], cbufs[b], csems[b]).wait()
            pltpu.make_async_copy(
                risk_hbm.at[pl.ds(off, CHUNK)], rbufs[b], rsems[b]).wait()

        start_full(0, 0)
        start_full(1, 1)

        def outer(g, _):
            k = 2 * g
            wait_full(k, 0)
            run_accum(cbuf0, rbuf0, CHUNK // L)

            @pl.when(k + 2 < fr)
            def _():
                start_full(k + 2, 0)

            wait_full(k + 1, 1)
            run_accum(cbuf1, rbuf1, CHUNK // L)

            @pl.when(k + 3 < fr)
            def _():
                start_full(k + 3, 1)

            return 0

        lax.fori_loop(0, fr // 2, outer, 0)

        moff = base_rem + wid * mch
        pltpu.async_copy(conf_hbm.at[pl.ds(moff, mch)],
                         cbuf1.at[pl.ds(0, mch)], csem1)
        pltpu.async_copy(risk_hbm.at[pl.ds(moff, mch)],
                         rbuf1.at[pl.ds(0, mch)], rsem1)
        wait_full(fr - 1, 0)
        run_accum(cbuf0, rbuf0, CHUNK // L)
        pltpu.make_async_copy(conf_hbm.at[pl.ds(moff, mch)],
                              cbuf1.at[pl.ds(0, mch)], csem1).wait()
        pltpu.make_async_copy(risk_hbm.at[pl.ds(moff, mch)],
                              rbuf1.at[pl.ds(0, mch)], rsem1).wait()
        run_accum(cbuf1, rbuf1, mch // L)

        if tail2:
            toff = base_rem + NW * mch

            @pl.when(wid == NW - 1)
            def _():
                pltpu.sync_copy(conf_hbm.at[pl.ds(toff, tail2)],
                                cbuf0.at[pl.ds(0, tail2)])
                pltpu.sync_copy(risk_hbm.at[pl.ds(toff, tail2)],
                                rbuf0.at[pl.ds(0, tail2)])
                run_accum(cbuf0, rbuf0, tail2 // L)

        base = wid * (3 * HIST_PAD)
        pltpu.sync_copy(hcnt, out_hbm.at[pl.ds(base, HIST_PAD)])
        pltpu.sync_copy(hconf, out_hbm.at[pl.ds(base + HIST_PAD, HIST_PAD)])
        pltpu.sync_copy(hrisk, out_hbm.at[pl.ds(base + 2 * HIST_PAD, HIST_PAD)])

    return sc_call


def kernel(confidences, risk):
    n = confidences.shape[0]
    p = _make_sc_call(n)(confidences, risk)
    s = p.reshape(NW, 3, 2, 128).sum(axis=0)
    s = s.reshape(3, HIST_PAD)[:, :HIST].reshape(3, N_BINS, L).sum(axis=2)
    cnt = s[0]
    sum_conf = s[1]
    sum_risk = cnt - s[2]
    safe = jnp.maximum(cnt, 1.0)
    contrib = jnp.abs(sum_conf / safe - sum_risk / safe) * (cnt / n)
    ece = jnp.sum(jnp.where(cnt > 0.0, contrib, 0.0))
    return ece.reshape((1,))
